# Initial kernel scaffold; baseline (speedup 1.0000x reference)
import jax
import jax.numpy as jnp
from jax.experimental import pallas as pl


def kernel(pred_scores, pred_bboxes, pred_dist, anchor_points, gt_bboxes, mask_gt, gt_labels):
    raise NotImplementedError("write your pallas kernel here")


# speedup vs baseline: 10.6424x; 10.6424x over previous
import functools

import jax
import jax.numpy as jnp
from jax.experimental import pallas as pl
from jax.experimental.pallas import tpu as pltpu

_TOPK = 13
_EPS = 1e-9
_CHUNK = 840


def _iota_f32(shape, dim):
    return jax.lax.broadcasted_iota(jnp.int32, shape, dim).astype(jnp.float32)


def _box_cols(ba_ref, sl):
    ba = ba_ref[0, sl, :]
    return (ba[:, 0:1], ba[:, 1:2], ba[:, 2:3], ba[:, 3:4],
            ba[:, 4:5], ba[:, 5:6])


def _chunk_iou(px1, py1, px2, py2, gx1, gy1, gx2, gy2, area_g):
    area_p = (px2 - px1) * (py2 - py1)
    iw = jnp.maximum(jnp.minimum(px2, gx2) - jnp.maximum(px1, gx1), 0.0)
    ih = jnp.maximum(jnp.minimum(py2, gy2) - jnp.maximum(py1, gy1), 0.0)
    inter = iw * ih
    return inter / (area_p + area_g - inter + _EPS)


def _loss_kernel(ps_ref, ba_ref, pd_ref, gtt_ref, labr_ref,
                 labc_ref, mg_ref, bce_ref, asc_ref, giou_ref, fg_ref,
                 dfl_ref, met_ref, *, reg_max):
    n, c = ps_ref.shape[1], ps_ref.shape[2]
    m = labr_ref.shape[2]
    nchunks = n // _CHUNK

    gtt = gtt_ref[0]
    labr = labr_ref[0]
    labc = labc_ref[0]
    mg = mg_ref[0]
    gx1 = gtt[0:1, :]
    gy1 = gtt[1:2, :]
    gx2 = gtt[2:3, :]
    gy2 = gtt[3:4, :]
    area_g = (gx2 - gx1) * (gy2 - gy1)
    w_cm = (_iota_f32((c, m), 0) == labr).astype(jnp.float32)
    w_mc = (_iota_f32((m, c), 1) == labc).astype(jnp.float32)

    def stage1(i, carry):
        sl = pl.ds(i * _CHUNK, _CHUNK)
        px1, py1, px2, py2, ax, ay = _box_cols(ba_ref, sl)
        dmin = jnp.minimum(jnp.minimum(ax - gx1, ay - gy1),
                           jnp.minimum(gx2 - ax, gy2 - ay))
        in_mask = (dmin > _EPS).astype(jnp.float32)
        iou = _chunk_iou(px1, py1, px2, py2, gx1, gy1, gx2, gy2, area_g)
        ps_gt = jax.lax.dot(ps_ref[0, sl, :], w_cm,
                            preferred_element_type=jnp.float32)
        iou2 = iou * iou
        met_ref[sl, :] = ps_gt * (iou2 * iou2 * iou2) * in_mask * mg
        return carry

    jax.lax.fori_loop(0, nchunks, stage1, 0)

    def extract_round(t, thresh):
        def body(i, run):
            mchunk = met_ref[pl.ds(i * _CHUNK, _CHUNK), :]
            masked = jnp.where(mchunk < thresh, mchunk, -1.0)
            return jnp.maximum(run, jnp.max(masked, axis=0, keepdims=True))
        return jax.lax.fori_loop(0, nchunks, body,
                                 jnp.full((1, m), -1.0, jnp.float32))

    thresh = jax.lax.fori_loop(0, _TOPK, extract_round,
                               jnp.full((1, m), 2.0, jnp.float32))

    def colmax(i, carry):
        cmet, ciou = carry
        sl = pl.ds(i * _CHUNK, _CHUNK)
        mchunk = met_ref[sl, :]
        keep = jnp.logical_and(mchunk >= thresh, mchunk > 0.0)
        px1, py1, px2, py2, ax, ay = _box_cols(ba_ref, sl)
        iou = _chunk_iou(px1, py1, px2, py2, gx1, gy1, gx2, gy2, area_g)
        cmet = jnp.maximum(cmet, jnp.max(jnp.where(keep, mchunk, 0.0),
                                         axis=0, keepdims=True))
        ciou = jnp.maximum(ciou, jnp.max(jnp.where(keep, iou, 0.0),
                                         axis=0, keepdims=True))
        return (cmet, ciou)

    zcol = jnp.zeros((1, m), jnp.float32)
    col_max_met, col_max_iou = jax.lax.fori_loop(0, nchunks, colmax,
                                                 (zcol, zcol))
    col_scale = col_max_iou / (col_max_met + _EPS)

    m_iota = _iota_f32((_CHUNK, m), 1)
    nb = reg_max + 1
    k_iota = _iota_f32((_CHUNK, nb), 1)

    def stage4(i, carry):
        s_bce, s_asc, s_giou, s_fg, s_dfl = carry
        sl = pl.ds(i * _CHUNK, _CHUNK)
        metric = met_ref[sl, :]
        keep = jnp.logical_and(metric >= thresh, metric > 0.0)
        topk = jnp.where(keep, metric, 0.0)
        row_max = jnp.max(topk, axis=1, keepdims=True)
        fg = (row_max > 0.0).astype(jnp.float32)
        idx = jnp.min(jnp.where(topk == row_max, m_iota, float(m)), axis=1,
                      keepdims=True)
        onehot = (m_iota == idx).astype(jnp.float32)
        abx1 = jnp.sum(onehot * gx1, axis=1, keepdims=True) * fg
        aby1 = jnp.sum(onehot * gy1, axis=1, keepdims=True) * fg
        abx2 = jnp.sum(onehot * gx2, axis=1, keepdims=True) * fg
        aby2 = jnp.sum(onehot * gy2, axis=1, keepdims=True) * fg
        norm = topk * col_scale

        ascore = jnp.zeros((_CHUNK, c), jnp.float32)
        for j in range(m):
            ascore = jnp.maximum(ascore, norm[:, j:j + 1] * w_mc[j:j + 1, :])

        p = jnp.clip(ps_ref[0, sl, :], 1e-9, 1.0 - 1e-9)
        pos = (ascore > 0.0).astype(jnp.float32)
        wgt = 0.75 * (p * p) * (1.0 - pos) + ascore
        bce = -(ascore * jnp.log(p) + (1.0 - ascore) * jnp.log(1.0 - p))
        s_bce = s_bce + jnp.sum(bce * wgt)
        s_asc = s_asc + jnp.sum(ascore)

        px1, py1, px2, py2, ax, ay = _box_cols(ba_ref, sl)
        area_p = (px2 - px1) * (py2 - py1)
        giw = jnp.maximum(jnp.minimum(px2, abx2) - jnp.maximum(px1, abx1), 0.0)
        gih = jnp.maximum(jnp.minimum(py2, aby2) - jnp.maximum(py1, aby1), 0.0)
        ginter = giw * gih
        garea2 = (abx2 - abx1) * (aby2 - aby1)
        gunion = area_p + garea2 - ginter
        enc = (jnp.maximum(px2, abx2) - jnp.minimum(px1, abx1)) * \
              (jnp.maximum(py2, aby2) - jnp.minimum(py1, aby1))
        giou = ginter / (gunion + _EPS) - (enc - gunion) / (enc + _EPS)
        s_giou = s_giou + jnp.sum((1.0 - giou) * fg)
        s_fg = s_fg + jnp.sum(fg)

        tgts = ((ax - abx1) / 8.0, (ay - aby1) / 8.0,
                (abx2 - ax) / 8.0, (aby2 - ay) / 8.0)
        dfl_acc = jnp.zeros((_CHUNK, 1), jnp.float32)
        for s, tgt_raw in enumerate(tgts):
            tgt = jnp.clip(tgt_raw, 0.0, reg_max - 0.01)
            tl = jnp.floor(tgt)
            wr = tgt - tl
            wl = 1.0 - wr
            pds = pd_ref[0, sl, s * nb:(s + 1) * nb]
            mx = jnp.max(pds, axis=1, keepdims=True)
            lse = jnp.log(jnp.sum(jnp.exp(pds - mx), axis=1,
                                  keepdims=True)) + mx
            sel_l = jnp.sum(jnp.where(k_iota == tl, pds, 0.0), axis=1,
                            keepdims=True)
            sel_r = jnp.sum(jnp.where(k_iota == tl + 1.0, pds, 0.0), axis=1,
                            keepdims=True)
            dfl_acc = dfl_acc + (lse - sel_l) * wl + (lse - sel_r) * wr
        s_dfl = s_dfl + jnp.sum(dfl_acc * 0.25 * fg)
        return (s_bce, s_asc, s_giou, s_fg, s_dfl)

    zero = jnp.float32(0.0)
    s_bce, s_asc, s_giou, s_fg, s_dfl = jax.lax.fori_loop(
        0, nchunks, stage4, (zero, zero, zero, zero, zero))

    bce_ref[0, :, :] = jnp.reshape(s_bce, (1, 1))
    asc_ref[0, :, :] = jnp.reshape(s_asc, (1, 1))
    giou_ref[0, :, :] = jnp.reshape(s_giou, (1, 1))
    fg_ref[0, :, :] = jnp.reshape(s_fg, (1, 1))
    dfl_ref[0, :, :] = jnp.reshape(s_dfl, (1, 1))


def kernel(pred_scores, pred_bboxes, pred_dist, anchor_points, gt_bboxes,
           mask_gt, gt_labels):
    B, N, C = pred_scores.shape
    M = gt_bboxes.shape[1]
    reg_max = pred_dist.shape[-1] // 4 - 1
    D = pred_dist.shape[-1]

    labels = jnp.clip(gt_labels[..., 0].astype(jnp.int32), 0, C - 1)
    labels = labels.astype(jnp.float32)
    lab_row = labels[:, None, :]
    lab_col = labels[:, :, None]
    gtt = jnp.transpose(gt_bboxes, (0, 2, 1))
    mg = jnp.transpose(mask_gt, (0, 2, 1))
    box_anch = jnp.concatenate(
        [pred_bboxes, jnp.broadcast_to(anchor_points[None], (B, N, 2))],
        axis=-1)

    scalar = jax.ShapeDtypeStruct((B, 1, 1), jnp.float32)
    out_spec = pl.BlockSpec((1, 1, 1), lambda b: (b, 0, 0))
    outs = pl.pallas_call(
        functools.partial(_loss_kernel, reg_max=reg_max),
        grid=(B,),
        in_specs=[
            pl.BlockSpec((1, N, C), lambda b: (b, 0, 0)),
            pl.BlockSpec((1, N, 6), lambda b: (b, 0, 0)),
            pl.BlockSpec((1, N, D), lambda b: (b, 0, 0)),
            pl.BlockSpec((1, 4, M), lambda b: (b, 0, 0)),
            pl.BlockSpec((1, 1, M), lambda b: (b, 0, 0)),
            pl.BlockSpec((1, M, 1), lambda b: (b, 0, 0)),
            pl.BlockSpec((1, 1, M), lambda b: (b, 0, 0)),
        ],
        out_specs=[out_spec] * 5,
        out_shape=[scalar] * 5,
        scratch_shapes=[
            pltpu.VMEM((N, M), jnp.float32),
        ],
    )(pred_scores, box_anch, pred_dist, gtt, lab_row, lab_col, mg)

    s_bce, s_asc, s_giou, s_fg, s_dfl = [jnp.sum(o) for o in outs]
    denom = jnp.maximum(s_asc, 1.0)
    num_fg = jnp.maximum(s_fg, 1.0)
    return s_bce / denom + 2.5 * s_giou / num_fg + 0.5 * s_dfl / num_fg


# speedup vs baseline: 58.6023x; 5.5065x over previous
import functools

import jax
import jax.numpy as jnp
from jax.experimental import pallas as pl
from jax.experimental.pallas import tpu as pltpu

_TOPK = 13
_EPS = 1e-9
_LANE = 128


def _iota_f32(shape, dim):
    return jax.lax.broadcasted_iota(jnp.int32, shape, dim).astype(jnp.float32)


def _shift_down(x, d, m):
    return jnp.concatenate(
        [jnp.zeros((d,) + x.shape[1:], x.dtype), x[:m - d]], axis=0)


def _shift_up(x, d, m):
    return jnp.concatenate(
        [x[d:], jnp.zeros((d,) + x.shape[1:], x.dtype)], axis=0)


def _loss_kernel(ps_ref, ba_ref, pd_ref, gt_ref, gtt_ref, labc_ref, labr_ref,
                 mg_ref, bce_ref, asc_ref, giou_ref, fg_ref, dfl_ref, met_ref,
                 *, reg_max, chunk):
    c, npad = ps_ref.shape[1], ps_ref.shape[2]
    m = labc_ref.shape[1]
    nchunks = npad // chunk

    gt = gt_ref[0]
    gtt = gtt_ref[0]
    labc = labc_ref[0]
    labr = labr_ref[0]
    mgc = mg_ref[0]
    gx1, gy1, gx2, gy2 = gt[:, 0:1], gt[:, 1:2], gt[:, 2:3], gt[:, 3:4]
    area_g = (gx2 - gx1) * (gy2 - gy1)
    w_mc = (_iota_f32((m, c), 1) == labc).astype(jnp.float32)
    w_cm = (_iota_f32((c, m), 0) == labr).astype(jnp.float32)

    def chunk_iou(pxy, sl):
        px1, py1, px2, py2 = pxy
        area_p = (px2 - px1) * (py2 - py1)
        iw = jnp.maximum(jnp.minimum(px2, gx2) - jnp.maximum(px1, gx1), 0.0)
        ih = jnp.maximum(jnp.minimum(py2, gy2) - jnp.maximum(py1, gy1), 0.0)
        inter = iw * ih
        return inter / (area_p + area_g - inter + _EPS)

    def box_rows(sl):
        ba = ba_ref[0, :, sl]
        return ((ba[0:1], ba[1:2], ba[2:3], ba[3:4]), ba[4:5], ba[5:6])

    def stage1(i, carry):
        sl = pl.ds(i * chunk, chunk)
        pxy, ax, ay = box_rows(sl)
        dmin = jnp.minimum(jnp.minimum(ax - gx1, ay - gy1),
                           jnp.minimum(gx2 - ax, gy2 - ay))
        in_mask = (dmin > _EPS).astype(jnp.float32)
        iou = chunk_iou(pxy, sl)
        ps_gt = jax.lax.dot(w_mc, ps_ref[0, :, sl],
                            preferred_element_type=jnp.float32)
        iou2 = iou * iou
        met_ref[:, sl] = ps_gt * (iou2 * iou2 * iou2) * in_mask * mgc
        return carry

    jax.lax.fori_loop(0, nchunks, stage1, 0)

    def extract_round(t, thresh):
        def body(i, run):
            mc = met_ref[:, pl.ds(i * chunk, chunk)]
            masked = jnp.where(mc < thresh, mc, -1.0)
            return jnp.maximum(run, jnp.max(masked, axis=1, keepdims=True))
        return jax.lax.fori_loop(0, nchunks, body,
                                 jnp.full((m, 1), -1.0, jnp.float32))

    thresh = jax.lax.fori_loop(0, _TOPK, extract_round,
                               jnp.full((m, 1), 2.0, jnp.float32))

    def colmax(i, carry):
        cmet, ciou = carry
        sl = pl.ds(i * chunk, chunk)
        mc = met_ref[:, sl]
        keep = jnp.logical_and(mc >= thresh, mc > 0.0)
        pxy, _, _ = box_rows(sl)
        iou = chunk_iou(pxy, sl)
        cmet = jnp.maximum(cmet, jnp.max(jnp.where(keep, mc, 0.0),
                                         axis=1, keepdims=True))
        ciou = jnp.maximum(ciou, jnp.max(jnp.where(keep, iou, 0.0),
                                         axis=1, keepdims=True))
        return (cmet, ciou)

    zcol = jnp.zeros((m, 1), jnp.float32)
    col_max_met, col_max_iou = jax.lax.fori_loop(0, nchunks, colmax,
                                                 (zcol, zcol))
    col_scale = col_max_iou / (col_max_met + _EPS)

    shifts = [1, 2, 4, 8, 16, 32]
    down_ok = [labc == _shift_down(labc, d, m) for d in shifts]
    up_ok = [labc == _shift_up(labc, d, m) for d in shifts]

    m_iota = _iota_f32((m, chunk), 0)
    nb = reg_max + 1
    k_iota = _iota_f32((nb, chunk), 0)

    def stage4(i, carry):
        s_bce, s_asc, s_giou, s_fg, s_dfl = carry
        sl = pl.ds(i * chunk, chunk)
        metric = met_ref[:, sl]
        keep = jnp.logical_and(metric >= thresh, metric > 0.0)
        topk = jnp.where(keep, metric, 0.0)
        row_max = jnp.max(topk, axis=0, keepdims=True)
        fg = (row_max > 0.0).astype(jnp.float32)
        idx = jnp.min(jnp.where(topk == row_max, m_iota, float(m)), axis=0,
                      keepdims=True)
        onehot = (m_iota == idx).astype(jnp.float32)
        ab = jax.lax.dot(gtt, onehot,
                         preferred_element_type=jnp.float32)
        abx1 = ab[0:1] * fg
        aby1 = ab[1:2] * fg
        abx2 = ab[2:3] * fg
        aby2 = ab[3:4] * fg
        norm = topk * col_scale

        pmax = norm
        for d, ok in zip(shifts, down_ok):
            pmax = jnp.maximum(pmax,
                               jnp.where(ok, _shift_down(pmax, d, m), 0.0))
        smax = norm
        for d, ok in zip(shifts, up_ok):
            smax = jnp.maximum(smax,
                               jnp.where(ok, _shift_up(smax, d, m), 0.0))
        gmax = jnp.maximum(pmax, smax)
        norm_win = jnp.where(
            jnp.logical_and(norm == gmax, norm > 0.0), norm, 0.0)
        ascore = jax.lax.dot(w_cm, norm_win,
                             preferred_element_type=jnp.float32)

        p = jnp.clip(ps_ref[0, :, sl], 1e-9, 1.0 - 1e-9)
        pos = (ascore > 0.0).astype(jnp.float32)
        wgt = 0.75 * (p * p) * (1.0 - pos) + ascore
        bce = -(ascore * jnp.log(p) + (1.0 - ascore) * jnp.log(1.0 - p))
        s_bce = s_bce + jnp.sum(bce * wgt)
        s_asc = s_asc + jnp.sum(ascore)

        pxy, ax, ay = box_rows(sl)
        px1, py1, px2, py2 = pxy
        area_p = (px2 - px1) * (py2 - py1)
        giw = jnp.maximum(jnp.minimum(px2, abx2) - jnp.maximum(px1, abx1), 0.0)
        gih = jnp.maximum(jnp.minimum(py2, aby2) - jnp.maximum(py1, aby1), 0.0)
        ginter = giw * gih
        garea2 = (abx2 - abx1) * (aby2 - aby1)
        gunion = area_p + garea2 - ginter
        enc = (jnp.maximum(px2, abx2) - jnp.minimum(px1, abx1)) * \
              (jnp.maximum(py2, aby2) - jnp.minimum(py1, aby1))
        giou = ginter / (gunion + _EPS) - (enc - gunion) / (enc + _EPS)
        s_giou = s_giou + jnp.sum((1.0 - giou) * fg)
        s_fg = s_fg + jnp.sum(fg)

        tgts = ((ax - abx1) / 8.0, (ay - aby1) / 8.0,
                (abx2 - ax) / 8.0, (aby2 - ay) / 8.0)
        dfl_acc = jnp.zeros((1, chunk), jnp.float32)
        for s, tgt_raw in enumerate(tgts):
            tgt = jnp.clip(tgt_raw, 0.0, reg_max - 0.01)
            tl = jnp.floor(tgt)
            wr = tgt - tl
            wl = 1.0 - wr
            pds = pd_ref[0, s, :, sl]
            mx = jnp.max(pds, axis=0, keepdims=True)
            lse = jnp.log(jnp.sum(jnp.exp(pds - mx), axis=0,
                                  keepdims=True)) + mx
            sel_l = jnp.sum(jnp.where(k_iota == tl, pds, 0.0), axis=0,
                            keepdims=True)
            sel_r = jnp.sum(jnp.where(k_iota == tl + 1.0, pds, 0.0), axis=0,
                            keepdims=True)
            dfl_acc = dfl_acc + (lse - sel_l) * wl + (lse - sel_r) * wr
        s_dfl = s_dfl + jnp.sum(dfl_acc * 0.25 * fg)
        return (s_bce, s_asc, s_giou, s_fg, s_dfl)

    zero = jnp.float32(0.0)
    s_bce, s_asc, s_giou, s_fg, s_dfl = jax.lax.fori_loop(
        0, nchunks, stage4, (zero, zero, zero, zero, zero))

    bce_ref[0, :, :] = jnp.reshape(s_bce, (1, 1))
    asc_ref[0, :, :] = jnp.reshape(s_asc, (1, 1))
    giou_ref[0, :, :] = jnp.reshape(s_giou, (1, 1))
    fg_ref[0, :, :] = jnp.reshape(s_fg, (1, 1))
    dfl_ref[0, :, :] = jnp.reshape(s_dfl, (1, 1))


def kernel(pred_scores, pred_bboxes, pred_dist, anchor_points, gt_bboxes,
           mask_gt, gt_labels):
    B, N, C = pred_scores.shape
    M = gt_bboxes.shape[1]
    reg_max = pred_dist.shape[-1] // 4 - 1
    nb = reg_max + 1

    npad = ((N + _LANE - 1) // _LANE) * _LANE
    chunk = npad // 6 if (npad // 6) % _LANE == 0 else npad
    pad = npad - N

    lab0 = jnp.clip(gt_labels[..., 0].astype(jnp.int32), 0, C - 1)
    order = jnp.argsort(lab0, axis=1)
    labs = jnp.take_along_axis(lab0, order, axis=1).astype(jnp.float32)
    gts = jnp.take_along_axis(gt_bboxes, order[..., None], axis=1)
    mgs = jnp.take_along_axis(mask_gt, order[..., None], axis=1)

    psT = jnp.pad(jnp.transpose(pred_scores, (0, 2, 1)),
                  ((0, 0), (0, 0), (0, pad)))
    baT = jnp.pad(jnp.concatenate(
        [jnp.transpose(pred_bboxes, (0, 2, 1)),
         jnp.broadcast_to(anchor_points.T[None], (B, 2, N))], axis=1),
        ((0, 0), (0, 0), (0, pad)))
    pdT = jnp.pad(jnp.transpose(pred_dist.reshape(B, N, 4, nb), (0, 2, 3, 1)),
                  ((0, 0), (0, 0), (0, 0), (0, pad)))
    gtt = jnp.transpose(gts, (0, 2, 1))
    labc = labs[:, :, None]
    labr = labs[:, None, :]

    scalar = jax.ShapeDtypeStruct((B, 1, 1), jnp.float32)
    out_spec = pl.BlockSpec((1, 1, 1), lambda b: (b, 0, 0))
    outs = pl.pallas_call(
        functools.partial(_loss_kernel, reg_max=reg_max, chunk=chunk),
        grid=(B,),
        in_specs=[
            pl.BlockSpec((1, C, npad), lambda b: (b, 0, 0)),
            pl.BlockSpec((1, 6, npad), lambda b: (b, 0, 0)),
            pl.BlockSpec((1, 4, nb, npad), lambda b: (b, 0, 0, 0)),
            pl.BlockSpec((1, M, 4), lambda b: (b, 0, 0)),
            pl.BlockSpec((1, 4, M), lambda b: (b, 0, 0)),
            pl.BlockSpec((1, M, 1), lambda b: (b, 0, 0)),
            pl.BlockSpec((1, 1, M), lambda b: (b, 0, 0)),
            pl.BlockSpec((1, M, 1), lambda b: (b, 0, 0)),
        ],
        out_specs=[out_spec] * 5,
        out_shape=[scalar] * 5,
        scratch_shapes=[
            pltpu.VMEM((M, npad), jnp.float32),
        ],
    )(psT, baT, pdT, gts, gtt, labc, labr, mgs)

    s_bce, s_asc, s_giou, s_fg, s_dfl = [jnp.sum(o) for o in outs]
    denom = jnp.maximum(s_asc, 1.0)
    num_fg = jnp.maximum(s_fg, 1.0)
    return s_bce / denom + 2.5 * s_giou / num_fg + 0.5 * s_dfl / num_fg


# speedup vs baseline: 61.3485x; 1.0469x over previous
import functools

import jax
import jax.numpy as jnp
from jax.experimental import pallas as pl
from jax.experimental.pallas import tpu as pltpu

_TOPK = 13
_EPS = 1e-9
_LANE = 128


def _iota_f32(shape, dim):
    return jax.lax.broadcasted_iota(jnp.int32, shape, dim).astype(jnp.float32)


def _shift_down(x, d, m):
    return jnp.concatenate(
        [jnp.zeros((d,) + x.shape[1:], x.dtype), x[:m - d]], axis=0)


def _shift_up(x, d, m):
    return jnp.concatenate(
        [x[d:], jnp.zeros((d,) + x.shape[1:], x.dtype)], axis=0)


def _loss_kernel(ps_ref, ba_ref, pd_ref, gt_ref, gtt_ref, labc_ref, labr_ref,
                 mg_ref, bce_ref, asc_ref, giou_ref, fg_ref, dfl_ref, met_ref,
                 iou_ref, *, reg_max, chunk):
    c, npad = ps_ref.shape[1], ps_ref.shape[2]
    m = labc_ref.shape[1]
    nchunks = npad // chunk

    gt = gt_ref[0]
    gtt = gtt_ref[0]
    labc = labc_ref[0]
    labr = labr_ref[0]
    mgc = mg_ref[0]
    gx1, gy1, gx2, gy2 = gt[:, 0:1], gt[:, 1:2], gt[:, 2:3], gt[:, 3:4]
    area_g = (gx2 - gx1) * (gy2 - gy1)
    w_mc = (_iota_f32((m, c), 1) == labc).astype(jnp.float32)
    w_cm = (_iota_f32((c, m), 0) == labr).astype(jnp.float32)

    def chunk_iou(pxy, sl):
        px1, py1, px2, py2 = pxy
        area_p = (px2 - px1) * (py2 - py1)
        iw = jnp.maximum(jnp.minimum(px2, gx2) - jnp.maximum(px1, gx1), 0.0)
        ih = jnp.maximum(jnp.minimum(py2, gy2) - jnp.maximum(py1, gy1), 0.0)
        inter = iw * ih
        return inter / (area_p + area_g - inter + _EPS)

    def box_rows(sl):
        ba = ba_ref[0, :, sl]
        return ((ba[0:1], ba[1:2], ba[2:3], ba[3:4]), ba[4:5], ba[5:6])

    def stage1(i, carry):
        sl = pl.ds(i * chunk, chunk)
        pxy, ax, ay = box_rows(sl)
        dmin = jnp.minimum(jnp.minimum(ax - gx1, ay - gy1),
                           jnp.minimum(gx2 - ax, gy2 - ay))
        in_mask = (dmin > _EPS).astype(jnp.float32)
        iou = chunk_iou(pxy, sl)
        iou_ref[:, sl] = iou
        ps_gt = jax.lax.dot(w_mc, ps_ref[0, :, sl],
                            preferred_element_type=jnp.float32)
        iou2 = iou * iou
        met_ref[:, sl] = ps_gt * (iou2 * iou2 * iou2) * in_mask * mgc
        return carry

    jax.lax.fori_loop(0, nchunks, stage1, 0)

    def extract_round(t, thresh):
        def body(i, run):
            mc = met_ref[:, pl.ds(i * chunk, chunk)]
            masked = jnp.where(mc < thresh, mc, -1.0)
            return jnp.maximum(run, jnp.max(masked, axis=1, keepdims=True))
        return jax.lax.fori_loop(0, nchunks, body,
                                 jnp.full((m, 1), -1.0, jnp.float32))

    thresh = jax.lax.fori_loop(0, _TOPK, extract_round,
                               jnp.full((m, 1), 2.0, jnp.float32))

    def colmax(i, carry):
        cmet, ciou = carry
        sl = pl.ds(i * chunk, chunk)
        mc = met_ref[:, sl]
        keep = jnp.logical_and(mc >= thresh, mc > 0.0)
        iou = iou_ref[:, sl]
        cmet = jnp.maximum(cmet, jnp.max(jnp.where(keep, mc, 0.0),
                                         axis=1, keepdims=True))
        ciou = jnp.maximum(ciou, jnp.max(jnp.where(keep, iou, 0.0),
                                         axis=1, keepdims=True))
        return (cmet, ciou)

    zcol = jnp.zeros((m, 1), jnp.float32)
    col_max_met, col_max_iou = jax.lax.fori_loop(0, nchunks, colmax,
                                                 (zcol, zcol))
    col_scale = col_max_iou / (col_max_met + _EPS)

    shifts = [1, 2, 4, 8, 16, 32]
    down_ok = [labc == _shift_down(labc, d, m) for d in shifts]
    up_ok = [labc == _shift_up(labc, d, m) for d in shifts]

    m_iota = _iota_f32((m, chunk), 0)
    nb = reg_max + 1
    k_iota = _iota_f32((nb, chunk), 0)

    def stage4(i, carry):
        s_bce, s_asc, s_giou, s_fg, s_dfl = carry
        sl = pl.ds(i * chunk, chunk)
        metric = met_ref[:, sl]
        keep = jnp.logical_and(metric >= thresh, metric > 0.0)
        topk = jnp.where(keep, metric, 0.0)
        row_max = jnp.max(topk, axis=0, keepdims=True)
        fg = (row_max > 0.0).astype(jnp.float32)
        idx = jnp.min(jnp.where(topk == row_max, m_iota, float(m)), axis=0,
                      keepdims=True)
        onehot = (m_iota == idx).astype(jnp.float32)
        ab = jax.lax.dot(gtt, onehot,
                         preferred_element_type=jnp.float32)
        abx1 = ab[0:1] * fg
        aby1 = ab[1:2] * fg
        abx2 = ab[2:3] * fg
        aby2 = ab[3:4] * fg
        norm = topk * col_scale

        pmax = norm
        for d, ok in zip(shifts, down_ok):
            pmax = jnp.maximum(pmax,
                               jnp.where(ok, _shift_down(pmax, d, m), 0.0))
        smax = norm
        for d, ok in zip(shifts, up_ok):
            smax = jnp.maximum(smax,
                               jnp.where(ok, _shift_up(smax, d, m), 0.0))
        gmax = jnp.maximum(pmax, smax)
        norm_win = jnp.where(
            jnp.logical_and(norm == gmax, norm > 0.0), norm, 0.0)
        ascore = jax.lax.dot(w_cm, norm_win,
                             preferred_element_type=jnp.float32)

        p = jnp.clip(ps_ref[0, :, sl], 1e-9, 1.0 - 1e-9)
        l1 = jnp.log(1.0 - p)
        a = ascore
        fg_term = a * (l1 + a * (jnp.log(p) - l1))
        bg_term = (0.75 * p * p) * l1
        s_bce = s_bce - jnp.sum(jnp.where(a > 0.0, fg_term, bg_term))
        s_asc = s_asc + jnp.sum(a)

        pxy, ax, ay = box_rows(sl)
        px1, py1, px2, py2 = pxy
        area_p = (px2 - px1) * (py2 - py1)
        giw = jnp.maximum(jnp.minimum(px2, abx2) - jnp.maximum(px1, abx1), 0.0)
        gih = jnp.maximum(jnp.minimum(py2, aby2) - jnp.maximum(py1, aby1), 0.0)
        ginter = giw * gih
        garea2 = (abx2 - abx1) * (aby2 - aby1)
        gunion = area_p + garea2 - ginter
        enc = (jnp.maximum(px2, abx2) - jnp.minimum(px1, abx1)) * \
              (jnp.maximum(py2, aby2) - jnp.minimum(py1, aby1))
        giou = ginter / (gunion + _EPS) - (enc - gunion) / (enc + _EPS)
        s_giou = s_giou + jnp.sum((1.0 - giou) * fg)
        s_fg = s_fg + jnp.sum(fg)

        tgts = ((ax - abx1) / 8.0, (ay - aby1) / 8.0,
                (abx2 - ax) / 8.0, (aby2 - ay) / 8.0)
        dfl_acc = jnp.zeros((1, chunk), jnp.float32)
        for s, tgt_raw in enumerate(tgts):
            tgt = jnp.clip(tgt_raw, 0.0, reg_max - 0.01)
            tl = jnp.floor(tgt)
            wr = tgt - tl
            wl = 1.0 - wr
            pds = pd_ref[0, s, :, sl]
            mx = jnp.max(pds, axis=0, keepdims=True)
            lse = jnp.log(jnp.sum(jnp.exp(pds - mx), axis=0,
                                  keepdims=True)) + mx
            sel_l = jnp.sum(jnp.where(k_iota == tl, pds, 0.0), axis=0,
                            keepdims=True)
            sel_r = jnp.sum(jnp.where(k_iota == tl + 1.0, pds, 0.0), axis=0,
                            keepdims=True)
            dfl_acc = dfl_acc + (lse - sel_l) * wl + (lse - sel_r) * wr
        s_dfl = s_dfl + jnp.sum(dfl_acc * 0.25 * fg)
        return (s_bce, s_asc, s_giou, s_fg, s_dfl)

    zero = jnp.float32(0.0)
    s_bce, s_asc, s_giou, s_fg, s_dfl = jax.lax.fori_loop(
        0, nchunks, stage4, (zero, zero, zero, zero, zero))

    bce_ref[0, :, :] = jnp.reshape(s_bce, (1, 1))
    asc_ref[0, :, :] = jnp.reshape(s_asc, (1, 1))
    giou_ref[0, :, :] = jnp.reshape(s_giou, (1, 1))
    fg_ref[0, :, :] = jnp.reshape(s_fg, (1, 1))
    dfl_ref[0, :, :] = jnp.reshape(s_dfl, (1, 1))


def kernel(pred_scores, pred_bboxes, pred_dist, anchor_points, gt_bboxes,
           mask_gt, gt_labels):
    B, N, C = pred_scores.shape
    M = gt_bboxes.shape[1]
    reg_max = pred_dist.shape[-1] // 4 - 1
    nb = reg_max + 1

    npad = ((N + _LANE - 1) // _LANE) * _LANE
    chunk = npad // 6 if (npad // 6) % _LANE == 0 else npad
    pad = npad - N

    lab0 = jnp.clip(gt_labels[..., 0].astype(jnp.int32), 0, C - 1)
    order = jnp.argsort(lab0, axis=1)
    labs = jnp.take_along_axis(lab0, order, axis=1).astype(jnp.float32)
    gts = jnp.take_along_axis(gt_bboxes, order[..., None], axis=1)
    mgs = jnp.take_along_axis(mask_gt, order[..., None], axis=1)

    psT = jnp.pad(jnp.transpose(pred_scores, (0, 2, 1)),
                  ((0, 0), (0, 0), (0, pad)))
    baT = jnp.pad(jnp.concatenate(
        [jnp.transpose(pred_bboxes, (0, 2, 1)),
         jnp.broadcast_to(anchor_points.T[None], (B, 2, N))], axis=1),
        ((0, 0), (0, 0), (0, pad)))
    pdT = jnp.pad(jnp.transpose(pred_dist.reshape(B, N, 4, nb), (0, 2, 3, 1)),
                  ((0, 0), (0, 0), (0, 0), (0, pad)))
    gtt = jnp.transpose(gts, (0, 2, 1))
    labc = labs[:, :, None]
    labr = labs[:, None, :]

    scalar = jax.ShapeDtypeStruct((B, 1, 1), jnp.float32)
    out_spec = pl.BlockSpec((1, 1, 1), lambda b: (b, 0, 0))
    outs = pl.pallas_call(
        functools.partial(_loss_kernel, reg_max=reg_max, chunk=chunk),
        grid=(B,),
        in_specs=[
            pl.BlockSpec((1, C, npad), lambda b: (b, 0, 0)),
            pl.BlockSpec((1, 6, npad), lambda b: (b, 0, 0)),
            pl.BlockSpec((1, 4, nb, npad), lambda b: (b, 0, 0, 0)),
            pl.BlockSpec((1, M, 4), lambda b: (b, 0, 0)),
            pl.BlockSpec((1, 4, M), lambda b: (b, 0, 0)),
            pl.BlockSpec((1, M, 1), lambda b: (b, 0, 0)),
            pl.BlockSpec((1, 1, M), lambda b: (b, 0, 0)),
            pl.BlockSpec((1, M, 1), lambda b: (b, 0, 0)),
        ],
        out_specs=[out_spec] * 5,
        out_shape=[scalar] * 5,
        scratch_shapes=[
            pltpu.VMEM((M, npad), jnp.float32),
            pltpu.VMEM((M, npad), jnp.float32),
        ],
    )(psT, baT, pdT, gts, gtt, labc, labr, mgs)

    s_bce, s_asc, s_giou, s_fg, s_dfl = [jnp.sum(o) for o in outs]
    denom = jnp.maximum(s_asc, 1.0)
    num_fg = jnp.maximum(s_fg, 1.0)
    return s_bce / denom + 2.5 * s_giou / num_fg + 0.5 * s_dfl / num_fg


# speedup vs baseline: 83.3027x; 1.3579x over previous
import functools

import jax
import jax.numpy as jnp
from jax.experimental import pallas as pl
from jax.experimental.pallas import tpu as pltpu

_TOPK = 13
_EPS = 1e-9
_LANE = 128


def _iota_f32(shape, dim):
    return jax.lax.broadcasted_iota(jnp.int32, shape, dim).astype(jnp.float32)


def _shift_down(x, d, m):
    return jnp.concatenate(
        [jnp.zeros((d,) + x.shape[1:], x.dtype), x[:m - d]], axis=0)


def _shift_up(x, d, m):
    return jnp.concatenate(
        [x[d:], jnp.zeros((d,) + x.shape[1:], x.dtype)], axis=0)


def _loss_kernel(ps_ref, ba_ref, pd_ref, gt_ref, gtt_ref, labc_ref, labr_ref,
                 mg_ref, bce_ref, asc_ref, giou_ref, fg_ref, dfl_ref, met_ref,
                 iou_ref, *, reg_max, chunk):
    c, npad = ps_ref.shape[1], ps_ref.shape[2]
    m = labc_ref.shape[1]
    nchunks = npad // chunk

    gt = gt_ref[0]
    gtt = gtt_ref[0]
    labc = labc_ref[0]
    labr = labr_ref[0]
    mgc = mg_ref[0]
    gx1, gy1, gx2, gy2 = gt[:, 0:1], gt[:, 1:2], gt[:, 2:3], gt[:, 3:4]
    area_g = (gx2 - gx1) * (gy2 - gy1)
    w_mc = (_iota_f32((m, c), 1) == labc).astype(jnp.float32)
    w_cm = (_iota_f32((c, m), 0) == labr).astype(jnp.float32)

    def chunk_iou(pxy, sl):
        px1, py1, px2, py2 = pxy
        area_p = (px2 - px1) * (py2 - py1)
        iw = jnp.maximum(jnp.minimum(px2, gx2) - jnp.maximum(px1, gx1), 0.0)
        ih = jnp.maximum(jnp.minimum(py2, gy2) - jnp.maximum(py1, gy1), 0.0)
        inter = iw * ih
        return inter / (area_p + area_g - inter + _EPS)

    def box_rows(sl):
        ba = ba_ref[0, :, sl]
        return ((ba[0:1], ba[1:2], ba[2:3], ba[3:4]), ba[4:5], ba[5:6])

    def stage1(i, carry):
        sl = pl.ds(i * chunk, chunk)
        pxy, ax, ay = box_rows(sl)
        dmin = jnp.minimum(jnp.minimum(ax - gx1, ay - gy1),
                           jnp.minimum(gx2 - ax, gy2 - ay))
        in_mask = (dmin > _EPS).astype(jnp.float32)
        iou = chunk_iou(pxy, sl)
        iou_ref[:, sl] = iou
        ps_gt = jax.lax.dot(w_mc, ps_ref[0, :, sl],
                            preferred_element_type=jnp.float32)
        iou2 = iou * iou
        met_ref[:, sl] = ps_gt * (iou2 * iou2 * iou2) * in_mask * mgc

    for i in range(nchunks):
        stage1(i, 0)

    thresh = jnp.full((m, 1), 2.0, jnp.float32)
    for _ in range(_TOPK):
        run = jnp.full((m, 1), -1.0, jnp.float32)
        for i in range(nchunks):
            mc = met_ref[:, pl.ds(i * chunk, chunk)]
            masked = jnp.where(mc < thresh, mc, -1.0)
            run = jnp.maximum(run, jnp.max(masked, axis=1, keepdims=True))
        thresh = run

    def colmax(i, carry):
        cmet, ciou = carry
        sl = pl.ds(i * chunk, chunk)
        mc = met_ref[:, sl]
        keep = jnp.logical_and(mc >= thresh, mc > 0.0)
        iou = iou_ref[:, sl]
        cmet = jnp.maximum(cmet, jnp.max(jnp.where(keep, mc, 0.0),
                                         axis=1, keepdims=True))
        ciou = jnp.maximum(ciou, jnp.max(jnp.where(keep, iou, 0.0),
                                         axis=1, keepdims=True))
        return (cmet, ciou)

    zcol = jnp.zeros((m, 1), jnp.float32)
    carry = (zcol, zcol)
    for i in range(nchunks):
        carry = colmax(i, carry)
    col_max_met, col_max_iou = carry
    col_scale = col_max_iou / (col_max_met + _EPS)

    shifts = [1, 2, 4, 8, 16, 32]
    down_ok = [labc == _shift_down(labc, d, m) for d in shifts]
    up_ok = [labc == _shift_up(labc, d, m) for d in shifts]

    nb = reg_max + 1
    k_iota = _iota_f32((nb, chunk), 0)

    def stage4(i, carry):
        s_bce, s_asc, s_giou, s_fg, s_dfl = carry
        sl = pl.ds(i * chunk, chunk)
        metric = met_ref[:, sl]
        keep = jnp.logical_and(metric >= thresh, metric > 0.0)
        topk = jnp.where(keep, metric, 0.0)
        row_max = jnp.max(topk, axis=0, keepdims=True)
        fg = (row_max > 0.0).astype(jnp.float32)
        onehot = jnp.where(jnp.logical_and(topk == row_max, row_max > 0.0),
                           1.0, 0.0)
        ab = jax.lax.dot(gtt, onehot,
                         preferred_element_type=jnp.float32)
        abx1 = ab[0:1]
        aby1 = ab[1:2]
        abx2 = ab[2:3]
        aby2 = ab[3:4]
        norm = topk * col_scale

        pmax = norm
        for d, ok in zip(shifts, down_ok):
            pmax = jnp.maximum(pmax,
                               jnp.where(ok, _shift_down(pmax, d, m), 0.0))
        smax = norm
        for d, ok in zip(shifts, up_ok):
            smax = jnp.maximum(smax,
                               jnp.where(ok, _shift_up(smax, d, m), 0.0))
        gmax = jnp.maximum(pmax, smax)
        norm_win = jnp.where(
            jnp.logical_and(norm == gmax, norm > 0.0), norm, 0.0)
        ascore = jax.lax.dot(w_cm, norm_win,
                             preferred_element_type=jnp.float32)

        p = jnp.clip(ps_ref[0, :, sl], 1e-9, 1.0 - 1e-9)
        a = ascore
        c1 = jnp.where(a > 0.0, a * (1.0 - a), 0.75 * (p * p))
        s_bce = s_bce - jnp.sum(c1 * jnp.log(1.0 - p) + (a * a) * jnp.log(p),
                                axis=0, keepdims=True)
        s_asc = s_asc + jnp.sum(a, axis=0, keepdims=True)

        pxy, ax, ay = box_rows(sl)
        px1, py1, px2, py2 = pxy
        area_p = (px2 - px1) * (py2 - py1)
        giw = jnp.maximum(jnp.minimum(px2, abx2) - jnp.maximum(px1, abx1), 0.0)
        gih = jnp.maximum(jnp.minimum(py2, aby2) - jnp.maximum(py1, aby1), 0.0)
        ginter = giw * gih
        garea2 = (abx2 - abx1) * (aby2 - aby1)
        gunion = area_p + garea2 - ginter
        enc = (jnp.maximum(px2, abx2) - jnp.minimum(px1, abx1)) * \
              (jnp.maximum(py2, aby2) - jnp.minimum(py1, aby1))
        giou = ginter / (gunion + _EPS) - (enc - gunion) / (enc + _EPS)
        s_giou = s_giou + (1.0 - giou) * fg
        s_fg = s_fg + fg

        tgts = ((ax - abx1) / 8.0, (ay - aby1) / 8.0,
                (abx2 - ax) / 8.0, (aby2 - ay) / 8.0)
        dfl_acc = jnp.zeros((1, chunk), jnp.float32)
        for s, tgt_raw in enumerate(tgts):
            tgt = jnp.clip(tgt_raw, 0.0, reg_max - 0.01)
            pds = pd_ref[0, s, :, sl]
            mx = jnp.max(pds, axis=0, keepdims=True)
            lse = jnp.log(jnp.sum(jnp.exp(pds - mx), axis=0,
                                  keepdims=True)) + mx
            wk = jnp.maximum(1.0 - jnp.abs(k_iota - tgt), 0.0)
            sel = jnp.sum(pds * wk, axis=0, keepdims=True)
            dfl_acc = dfl_acc + lse - sel
        s_dfl = s_dfl + dfl_acc * 0.25 * fg
        return (s_bce, s_asc, s_giou, s_fg, s_dfl)

    zrow = jnp.zeros((1, chunk), jnp.float32)
    carry4 = (zrow, zrow, zrow, zrow, zrow)
    for i in range(nchunks):
        carry4 = stage4(i, carry4)
    s_bce, s_asc, s_giou, s_fg, s_dfl = carry4

    bce_ref[0, :, :] = jnp.reshape(jnp.sum(s_bce), (1, 1))
    asc_ref[0, :, :] = jnp.reshape(jnp.sum(s_asc), (1, 1))
    giou_ref[0, :, :] = jnp.reshape(jnp.sum(s_giou), (1, 1))
    fg_ref[0, :, :] = jnp.reshape(jnp.sum(s_fg), (1, 1))
    dfl_ref[0, :, :] = jnp.reshape(jnp.sum(s_dfl), (1, 1))


def kernel(pred_scores, pred_bboxes, pred_dist, anchor_points, gt_bboxes,
           mask_gt, gt_labels):
    B, N, C = pred_scores.shape
    M = gt_bboxes.shape[1]
    reg_max = pred_dist.shape[-1] // 4 - 1
    nb = reg_max + 1

    npad = ((N + _LANE - 1) // _LANE) * _LANE
    chunk = npad // 6 if (npad // 6) % _LANE == 0 else npad
    pad = npad - N

    lab0 = jnp.clip(gt_labels[..., 0].astype(jnp.int32), 0, C - 1)
    order = jnp.argsort(lab0, axis=1)
    labs = jnp.take_along_axis(lab0, order, axis=1).astype(jnp.float32)
    gts = jnp.take_along_axis(gt_bboxes, order[..., None], axis=1)
    mgs = jnp.take_along_axis(mask_gt, order[..., None], axis=1)

    psT = jnp.pad(jnp.transpose(pred_scores, (0, 2, 1)),
                  ((0, 0), (0, 0), (0, pad)))
    baT = jnp.pad(jnp.concatenate(
        [jnp.transpose(pred_bboxes, (0, 2, 1)),
         jnp.broadcast_to(anchor_points.T[None], (B, 2, N))], axis=1),
        ((0, 0), (0, 0), (0, pad)))
    pdT = jnp.pad(jnp.transpose(pred_dist.reshape(B, N, 4, nb), (0, 2, 3, 1)),
                  ((0, 0), (0, 0), (0, 0), (0, pad)))
    gtt = jnp.transpose(gts, (0, 2, 1))
    labc = labs[:, :, None]
    labr = labs[:, None, :]

    scalar = jax.ShapeDtypeStruct((B, 1, 1), jnp.float32)
    out_spec = pl.BlockSpec((1, 1, 1), lambda b: (b, 0, 0))
    outs = pl.pallas_call(
        functools.partial(_loss_kernel, reg_max=reg_max, chunk=chunk),
        grid=(B,),
        in_specs=[
            pl.BlockSpec((1, C, npad), lambda b: (b, 0, 0)),
            pl.BlockSpec((1, 6, npad), lambda b: (b, 0, 0)),
            pl.BlockSpec((1, 4, nb, npad), lambda b: (b, 0, 0, 0)),
            pl.BlockSpec((1, M, 4), lambda b: (b, 0, 0)),
            pl.BlockSpec((1, 4, M), lambda b: (b, 0, 0)),
            pl.BlockSpec((1, M, 1), lambda b: (b, 0, 0)),
            pl.BlockSpec((1, 1, M), lambda b: (b, 0, 0)),
            pl.BlockSpec((1, M, 1), lambda b: (b, 0, 0)),
        ],
        out_specs=[out_spec] * 5,
        out_shape=[scalar] * 5,
        scratch_shapes=[
            pltpu.VMEM((M, npad), jnp.float32),
            pltpu.VMEM((M, npad), jnp.float32),
        ],
    )(psT, baT, pdT, gts, gtt, labc, labr, mgs)

    s_bce, s_asc, s_giou, s_fg, s_dfl = [jnp.sum(o) for o in outs]
    denom = jnp.maximum(s_asc, 1.0)
    num_fg = jnp.maximum(s_fg, 1.0)
    return s_bce / denom + 2.5 * s_giou / num_fg + 0.5 * s_dfl / num_fg
